# baseline (speedup 1.0000x reference)
import functools

import jax
import jax.numpy as jnp
from jax import lax
from jax.experimental import pallas as pl
from jax.experimental.pallas import tpu as pltpu

NS, NT, D = 4096, 8192, 512


def _y0_kernel(inp_t_ref, w_ref, y0_ref):
    y0 = lax.dot_general(inp_t_ref[...].astype(jnp.bfloat16),
                         w_ref[...].astype(jnp.bfloat16),
                         (((1,), (0,)), ((), ())),
                         preferred_element_type=jnp.float32)
    y0_ref[...] = y0.astype(jnp.bfloat16)


def _y_side(inp_t, W):
    bt = 2048
    return pl.pallas_call(
        _y0_kernel,
        grid=(NT // bt,),
        in_specs=[
            pl.BlockSpec((bt, D), lambda t: (t, 0)),
            pl.BlockSpec((D, D), lambda t: (0, 0)),
        ],
        out_specs=pl.BlockSpec((bt, D), lambda t: (t, 0)),
        out_shape=jax.ShapeDtypeStruct((NT, D), jnp.bfloat16),
        compiler_params=pltpu.CompilerParams(
            dimension_semantics=("parallel",)),
    )(inp_t, W)


BM = 512
SK = 1024
GI = NS // BM
GK = NT // SK


def _p1_kernel(adj_ref, y0b_ref, inp_s_ref, w_ref,
               adjb_ref, x_ref, xst_ref, irdt_ref,
               slab_ref, acc_ref, ys_ref, rowsum_ref, colsum_ref):
    k = pl.program_id(0)
    i = pl.program_id(1)

    @pl.when(k < GK)
    def _():
        blk = adj_ref[...]
        b16 = blk.astype(jnp.bfloat16)
        adjb_ref[...] = b16
        slab_ref[lax.rem(k, 2), pl.ds(i * BM, BM), :] = b16
        sr = jnp.sum(blk, axis=1, keepdims=True)
        sc = jnp.sum(blk, axis=0, keepdims=True)

        @pl.when(k == 0)
        def _():
            rowsum_ref[pl.ds(i * BM, BM), :] = sr

        @pl.when(k > 0)
        def _():
            rowsum_ref[pl.ds(i * BM, BM), :] += sr

        @pl.when(i == 0)
        def _():
            colsum_ref[:, pl.ds(k * SK, SK)] = sc

        @pl.when(i > 0)
        def _():
            colsum_ref[:, pl.ds(k * SK, SK)] += sc

    @pl.when(k >= 1)
    def _():
        @pl.when(i == 0)
        def _():
            cs = colsum_ref[:, pl.ds((k - 1) * SK, SK)]
            irdt_col = jnp.transpose(lax.rsqrt(cs + 1.0))
            irdt_ref[...] = irdt_col
            y0s = y0b_ref[...].astype(jnp.float32)
            ys_ref[...] = (y0s * irdt_col).astype(jnp.bfloat16)

        part = lax.dot_general(
            slab_ref[lax.rem(k - 1, 2), pl.ds(i * BM, BM), :], ys_ref[...],
            (((1,), (0,)), ((), ())), preferred_element_type=jnp.float32)

        @pl.when(k == 1)
        def _():
            acc_ref[pl.ds(i * BM, BM), :] = part

        @pl.when(k > 1)
        def _():
            acc_ref[pl.ds(i * BM, BM), :] += part

    @pl.when(k == GK)
    def _():
        x0 = lax.dot_general(inp_s_ref[...].astype(jnp.bfloat16),
                             w_ref[...].astype(jnp.bfloat16),
                             (((1,), (0,)), ((), ())),
                             preferred_element_type=jnp.float32)
        rsq = rowsum_ref[pl.ds(i * BM, BM), :] + 1.0
        t = jax.nn.relu(acc_ref[pl.ds(i * BM, BM), :] + x0)
        x_ref[...] = t * lax.rsqrt(rsq)
        xst_ref[...] = jnp.transpose((t * (1.0 / rsq)).astype(jnp.bfloat16))


def _p1(adj, y0b, inp_s, W):
    return pl.pallas_call(
        _p1_kernel,
        grid=(GK + 1, GI),
        in_specs=[
            pl.BlockSpec((BM, SK),
                         lambda k, i: (jnp.where(k == GK, GI - 1, i),
                                       jnp.minimum(k, GK - 1))),
            pl.BlockSpec((SK, D),
                         lambda k, i: (jnp.maximum(k - 1, 0), 0)),
            pl.BlockSpec((BM, D),
                         lambda k, i: (jnp.where(k == GK, i, 0), 0)),
            pl.BlockSpec((D, D), lambda k, i: (0, 0)),
        ],
        out_specs=[
            pl.BlockSpec((BM, SK),
                         lambda k, i: (jnp.where(k == GK, GI - 1, i),
                                       jnp.minimum(k, GK - 1))),
            pl.BlockSpec((BM, D),
                         lambda k, i: (jnp.where(k == GK, i, 0), 0)),
            pl.BlockSpec((D, BM),
                         lambda k, i: (0, jnp.where(k == GK, i, 0))),
            pl.BlockSpec((SK, 1),
                         lambda k, i: (jnp.maximum(k - 1, 0), 0)),
        ],
        out_shape=[
            jax.ShapeDtypeStruct((NS, NT), jnp.bfloat16),
            jax.ShapeDtypeStruct((NS, D), jnp.float32),
            jax.ShapeDtypeStruct((D, NS), jnp.bfloat16),
            jax.ShapeDtypeStruct((NT, 1), jnp.float32),
        ],
        scratch_shapes=[
            pltpu.VMEM((2, NS, SK), jnp.bfloat16),
            pltpu.VMEM((NS, D), jnp.float32),
            pltpu.VMEM((SK, D), jnp.bfloat16),
            pltpu.VMEM((NS, 1), jnp.float32),
            pltpu.VMEM((1, NT), jnp.float32),
        ],
        compiler_params=pltpu.CompilerParams(
            dimension_semantics=("arbitrary", "arbitrary")),
    )(adj, y0b, inp_s, W)


def _p2_kernel(adjb_ref, xst_ref, y0b_ref, irdt_ref, y_ref):
    tt = lax.dot_general(xst_ref[...], adjb_ref[...],
                         (((1,), (0,)), ((), ())),
                         preferred_element_type=jnp.float32)
    t = jnp.transpose(tt)
    y0 = y0b_ref[...].astype(jnp.float32)
    y_ref[...] = jax.nn.relu(y0 + t) * irdt_ref[...]


def _p2(adjb, xst, y0b, irdt):
    bn = 1024
    return pl.pallas_call(
        _p2_kernel,
        grid=(NT // bn,),
        in_specs=[
            pl.BlockSpec((NS, bn), lambda j: (0, j)),
            pl.BlockSpec((D, NS), lambda j: (0, 0)),
            pl.BlockSpec((bn, D), lambda j: (j, 0)),
            pl.BlockSpec((bn, 1), lambda j: (j, 0)),
        ],
        out_specs=pl.BlockSpec((bn, D), lambda j: (j, 0)),
        out_shape=jax.ShapeDtypeStruct((NT, D), jnp.float32),
        compiler_params=pltpu.CompilerParams(
            dimension_semantics=("parallel",)),
    )(adjb, xst, y0b, irdt)


def kernel(inp_s, inp_t, adj, W):
    y0b = _y_side(inp_t, W)
    adjb, x_out, xst, irdt = _p1(adj, y0b, inp_s, W)
    y_out = _p2(adjb, xst, y0b, irdt)
    return (x_out, y_out)


# speedup vs baseline: 1.1405x; 1.1405x over previous
import functools

import jax
import jax.numpy as jnp
from jax import lax
from jax.experimental import pallas as pl
from jax.experimental.pallas import tpu as pltpu

NS, NT, D = 4096, 8192, 512


def _y0_kernel(inp_t_ref, inp_s_ref, w_ref, y0_ref, x0_ref):
    t = pl.program_id(0)
    w16 = w_ref[...].astype(jnp.bfloat16)
    y0 = lax.dot_general(inp_t_ref[...].astype(jnp.bfloat16), w16,
                         (((1,), (0,)), ((), ())),
                         preferred_element_type=jnp.float32)
    y0_ref[...] = y0.astype(jnp.bfloat16)

    @pl.when(t < 2)
    def _():
        x0 = lax.dot_general(inp_s_ref[...].astype(jnp.bfloat16), w16,
                             (((1,), (0,)), ((), ())),
                             preferred_element_type=jnp.float32)
        x0_ref[...] = x0.astype(jnp.bfloat16)


def _y_side(inp_t, inp_s, W):
    bt = 2048
    return pl.pallas_call(
        _y0_kernel,
        grid=(NT // bt,),
        in_specs=[
            pl.BlockSpec((bt, D), lambda t: (t, 0)),
            pl.BlockSpec((bt, D), lambda t: (jnp.minimum(t, 1), 0)),
            pl.BlockSpec((D, D), lambda t: (0, 0)),
        ],
        out_specs=[
            pl.BlockSpec((bt, D), lambda t: (t, 0)),
            pl.BlockSpec((bt, D), lambda t: (jnp.minimum(t, 1), 0)),
        ],
        out_shape=[
            jax.ShapeDtypeStruct((NT, D), jnp.bfloat16),
            jax.ShapeDtypeStruct((NS, D), jnp.bfloat16),
        ],
        compiler_params=pltpu.CompilerParams(
            dimension_semantics=("arbitrary",)),
    )(inp_t, inp_s, W)


BM = 1024
SK = 1024
GI = NS // BM
GK = NT // SK


def _p1_kernel(adj_ref, y0b_ref, x0b_ref,
               adjb_ref, x_ref, xst_ref, irdt_ref,
               slab_ref, acc_ref, ys_ref, rowsum_ref, colsum_ref):
    k = pl.program_id(0)
    i = pl.program_id(1)

    @pl.when(k < GK)
    def _():
        blk = adj_ref[...]
        b16 = blk.astype(jnp.bfloat16)
        adjb_ref[...] = b16
        slab_ref[lax.rem(k, 2), pl.ds(i * BM, BM), :] = b16
        sr = jnp.sum(blk, axis=1, keepdims=True)
        sc = jnp.sum(blk, axis=0, keepdims=True)

        @pl.when(k == 0)
        def _():
            rowsum_ref[pl.ds(i * BM, BM), :] = sr

        @pl.when(k > 0)
        def _():
            rowsum_ref[pl.ds(i * BM, BM), :] += sr

        @pl.when(i == 0)
        def _():
            colsum_ref[:, pl.ds(k * SK, SK)] = sc

        @pl.when(i > 0)
        def _():
            colsum_ref[:, pl.ds(k * SK, SK)] += sc

    @pl.when(k >= 1)
    def _():
        @pl.when(i == 0)
        def _():
            cs = colsum_ref[:, pl.ds((k - 1) * SK, SK)]
            irdt_col = jnp.transpose(lax.rsqrt(cs + 1.0))
            irdt_ref[...] = irdt_col
            y0s = y0b_ref[...].astype(jnp.float32)
            ys_ref[...] = (y0s * irdt_col).astype(jnp.bfloat16)

        part = lax.dot_general(
            slab_ref[lax.rem(k - 1, 2), pl.ds(i * BM, BM), :], ys_ref[...],
            (((1,), (0,)), ((), ())), preferred_element_type=jnp.float32)

        @pl.when(k == 1)
        def _():
            acc_ref[pl.ds(i * BM, BM), :] = part

        @pl.when(k > 1)
        def _():
            acc_ref[pl.ds(i * BM, BM), :] += part

    @pl.when(k == GK)
    def _():
        x0 = x0b_ref[...].astype(jnp.float32)
        rsq = rowsum_ref[pl.ds(i * BM, BM), :] + 1.0
        t = jax.nn.relu(acc_ref[pl.ds(i * BM, BM), :] + x0)
        x_ref[...] = t * lax.rsqrt(rsq)
        xst_ref[...] = jnp.transpose((t * (1.0 / rsq)).astype(jnp.bfloat16))


def _p1(adj, y0b, x0b):
    return pl.pallas_call(
        _p1_kernel,
        grid=(GK + 1, GI),
        in_specs=[
            pl.BlockSpec((BM, SK),
                         lambda k, i: (jnp.where(k == GK, GI - 1, i),
                                       jnp.minimum(k, GK - 1))),
            pl.BlockSpec((SK, D),
                         lambda k, i: (jnp.maximum(k - 1, 0), 0)),
            pl.BlockSpec((BM, D),
                         lambda k, i: (jnp.where(k == GK, i, 0), 0)),
        ],
        out_specs=[
            pl.BlockSpec((BM, SK),
                         lambda k, i: (jnp.where(k == GK, GI - 1, i),
                                       jnp.minimum(k, GK - 1))),
            pl.BlockSpec((BM, D),
                         lambda k, i: (jnp.where(k == GK, i, 0), 0)),
            pl.BlockSpec((D, BM),
                         lambda k, i: (0, jnp.where(k == GK, i, 0))),
            pl.BlockSpec((SK, 1),
                         lambda k, i: (jnp.maximum(k - 1, 0), 0)),
        ],
        out_shape=[
            jax.ShapeDtypeStruct((NS, NT), jnp.bfloat16),
            jax.ShapeDtypeStruct((NS, D), jnp.float32),
            jax.ShapeDtypeStruct((D, NS), jnp.bfloat16),
            jax.ShapeDtypeStruct((NT, 1), jnp.float32),
        ],
        scratch_shapes=[
            pltpu.VMEM((2, NS, SK), jnp.bfloat16),
            pltpu.VMEM((NS, D), jnp.float32),
            pltpu.VMEM((SK, D), jnp.bfloat16),
            pltpu.VMEM((NS, 1), jnp.float32),
            pltpu.VMEM((1, NT), jnp.float32),
        ],
        compiler_params=pltpu.CompilerParams(
            dimension_semantics=("arbitrary", "arbitrary")),
    )(adj, y0b, x0b)


def _p2_kernel(adjb_ref, xst_ref, y0b_ref, irdt_ref, y_ref):
    tt = lax.dot_general(xst_ref[...], adjb_ref[...],
                         (((1,), (0,)), ((), ())),
                         preferred_element_type=jnp.float32)
    t = jnp.transpose(tt)
    y0 = y0b_ref[...].astype(jnp.float32)
    y_ref[...] = jax.nn.relu(y0 + t) * irdt_ref[...]


def _p2(adjb, xst, y0b, irdt):
    bn = 1024
    return pl.pallas_call(
        _p2_kernel,
        grid=(NT // bn,),
        in_specs=[
            pl.BlockSpec((NS, bn), lambda j: (0, j)),
            pl.BlockSpec((D, NS), lambda j: (0, 0)),
            pl.BlockSpec((bn, D), lambda j: (j, 0)),
            pl.BlockSpec((bn, 1), lambda j: (j, 0)),
        ],
        out_specs=pl.BlockSpec((bn, D), lambda j: (j, 0)),
        out_shape=jax.ShapeDtypeStruct((NT, D), jnp.float32),
        compiler_params=pltpu.CompilerParams(
            dimension_semantics=("parallel",)),
    )(adjb, xst, y0b, irdt)


def kernel(inp_s, inp_t, adj, W):
    y0b, x0b = _y_side(inp_t, inp_s, W)
    adjb, x_out, xst, irdt = _p1(adj, y0b, x0b)
    y_out = _p2(adjb, xst, y0b, irdt)
    return (x_out, y_out)


# speedup vs baseline: 1.1536x; 1.0115x over previous
import jax
import jax.numpy as jnp
from jax import lax
from jax.experimental import pallas as pl
from jax.experimental.pallas import tpu as pltpu

NS, NT, D = 4096, 8192, 512


def _y0_kernel(inp_t_ref, w_ref, y0_ref):
    y0 = lax.dot_general(inp_t_ref[...].astype(jnp.bfloat16),
                         w_ref[...].astype(jnp.bfloat16),
                         (((1,), (0,)), ((), ())),
                         preferred_element_type=jnp.float32)
    y0_ref[...] = y0.astype(jnp.bfloat16)


def _y_side(inp_t, W):
    bt = 2048
    return pl.pallas_call(
        _y0_kernel,
        grid=(NT // bt,),
        in_specs=[
            pl.BlockSpec((bt, D), lambda t: (t, 0)),
            pl.BlockSpec((D, D), lambda t: (0, 0)),
        ],
        out_specs=pl.BlockSpec((bt, D), lambda t: (t, 0)),
        out_shape=jax.ShapeDtypeStruct((NT, D), jnp.bfloat16),
        compiler_params=pltpu.CompilerParams(
            dimension_semantics=("parallel",)),
    )(inp_t, W)


BM = 1024
SK = 1024
GI = NS // BM
GK = NT // SK


def _p1_kernel(adj_ref, y0b_ref, inp_s_ref, w_ref,
               adjb_ref, x_ref, xst_ref, irdt_ref,
               slab_ref, acc_ref, ys_ref, rowsum_ref, colsum_ref):
    k = pl.program_id(0)
    i = pl.program_id(1)

    @pl.when(k < GK)
    def _():
        blk = adj_ref[...]
        b16 = blk.astype(jnp.bfloat16)
        adjb_ref[...] = b16
        slab_ref[lax.rem(k, 2), pl.ds(i * BM, BM), :] = b16
        sr = jnp.sum(blk, axis=1, keepdims=True)
        sc = jnp.sum(blk, axis=0, keepdims=True)

        @pl.when(k == 0)
        def _():
            rowsum_ref[pl.ds(i * BM, BM), :] = sr

        @pl.when(k > 0)
        def _():
            rowsum_ref[pl.ds(i * BM, BM), :] += sr

        @pl.when(i == 0)
        def _():
            colsum_ref[:, pl.ds(k * SK, SK)] = sc

        @pl.when(i > 0)
        def _():
            colsum_ref[:, pl.ds(k * SK, SK)] += sc

    @pl.when(k >= 1)
    def _():
        @pl.when(i == 0)
        def _():
            cs = colsum_ref[:, pl.ds((k - 1) * SK, SK)]
            irdt_col = jnp.transpose(lax.rsqrt(cs + 1.0))
            irdt_ref[...] = irdt_col
            y0s = y0b_ref[...].astype(jnp.float32)
            ys_ref[...] = (y0s * irdt_col).astype(jnp.bfloat16)

        part = lax.dot_general(
            slab_ref[lax.rem(k - 1, 2), pl.ds(i * BM, BM), :], ys_ref[...],
            (((1,), (0,)), ((), ())), preferred_element_type=jnp.float32)

        @pl.when(k == 1)
        def _():
            acc_ref[pl.ds(i * BM, BM), :] = part

        @pl.when(k > 1)
        def _():
            acc_ref[pl.ds(i * BM, BM), :] += part

    @pl.when(k == GK)
    def _():
        x0 = lax.dot_general(inp_s_ref[...].astype(jnp.bfloat16),
                             w_ref[...].astype(jnp.bfloat16),
                             (((1,), (0,)), ((), ())),
                             preferred_element_type=jnp.float32)
        rsq = rowsum_ref[pl.ds(i * BM, BM), :] + 1.0
        t = jax.nn.relu(acc_ref[pl.ds(i * BM, BM), :] + x0)
        x_ref[...] = t * lax.rsqrt(rsq)
        xst_ref[...] = jnp.transpose((t * (1.0 / rsq)).astype(jnp.bfloat16))


def _p1(adj, y0b, inp_s, W):
    return pl.pallas_call(
        _p1_kernel,
        grid=(GK + 1, GI),
        in_specs=[
            pl.BlockSpec((BM, SK),
                         lambda k, i: (jnp.where(k == GK, GI - 1, i),
                                       jnp.minimum(k, GK - 1))),
            pl.BlockSpec((SK, D),
                         lambda k, i: (jnp.maximum(k - 1, 0), 0)),
            pl.BlockSpec((BM, D),
                         lambda k, i: (jnp.where(k == GK, i, 0), 0)),
            pl.BlockSpec((D, D), lambda k, i: (0, 0)),
        ],
        out_specs=[
            pl.BlockSpec((BM, SK),
                         lambda k, i: (jnp.where(k == GK, GI - 1, i),
                                       jnp.minimum(k, GK - 1))),
            pl.BlockSpec((BM, D),
                         lambda k, i: (jnp.where(k == GK, i, 0), 0)),
            pl.BlockSpec((D, BM),
                         lambda k, i: (0, jnp.where(k == GK, i, 0))),
            pl.BlockSpec((SK, 1),
                         lambda k, i: (jnp.maximum(k - 1, 0), 0)),
        ],
        out_shape=[
            jax.ShapeDtypeStruct((NS, NT), jnp.bfloat16),
            jax.ShapeDtypeStruct((NS, D), jnp.float32),
            jax.ShapeDtypeStruct((D, NS), jnp.bfloat16),
            jax.ShapeDtypeStruct((NT, 1), jnp.float32),
        ],
        scratch_shapes=[
            pltpu.VMEM((2, NS, SK), jnp.bfloat16),
            pltpu.VMEM((NS, D), jnp.float32),
            pltpu.VMEM((SK, D), jnp.bfloat16),
            pltpu.VMEM((NS, 1), jnp.float32),
            pltpu.VMEM((1, NT), jnp.float32),
        ],
        compiler_params=pltpu.CompilerParams(
            dimension_semantics=("arbitrary", "arbitrary")),
    )(adj, y0b, inp_s, W)


def _p2_kernel(adjb_ref, xst_ref, y0b_ref, irdt_ref, y_ref):
    tt = lax.dot_general(xst_ref[...], adjb_ref[...],
                         (((1,), (0,)), ((), ())),
                         preferred_element_type=jnp.float32)
    t = jnp.transpose(tt)
    y0 = y0b_ref[...].astype(jnp.float32)
    y_ref[...] = jax.nn.relu(y0 + t) * irdt_ref[...]


def _p2(adjb, xst, y0b, irdt):
    bn = 1024
    return pl.pallas_call(
        _p2_kernel,
        grid=(NT // bn,),
        in_specs=[
            pl.BlockSpec((NS, bn), lambda j: (0, j)),
            pl.BlockSpec((D, NS), lambda j: (0, 0)),
            pl.BlockSpec((bn, D), lambda j: (j, 0)),
            pl.BlockSpec((bn, 1), lambda j: (j, 0)),
        ],
        out_specs=pl.BlockSpec((bn, D), lambda j: (j, 0)),
        out_shape=jax.ShapeDtypeStruct((NT, D), jnp.float32),
        compiler_params=pltpu.CompilerParams(
            dimension_semantics=("parallel",)),
    )(adjb, xst, y0b, irdt)


def kernel(inp_s, inp_t, adj, W):
    y0b = _y_side(inp_t, W)
    adjb, x_out, xst, irdt = _p1(adj, y0b, inp_s, W)
    y_out = _p2(adjb, xst, y0b, irdt)
    return (x_out, y_out)
